# baseline (speedup 1.0000x reference)
import functools

import jax
import jax.numpy as jnp
from jax import lax
from jax.experimental import pallas as pl
from jax.experimental.pallas import tpu as pltpu
from jax.experimental.pallas import tpu_sc as plsc

_NUM_CORES = 2
_NUM_SUBCORES = 16
_LANES = 16


@functools.partial(jax.jit, static_argnames=("seq_len", "chunk"))
def _embedding_add(idx, W_words, W_pos, seq_len, chunk):
    n = idx.shape[0]
    d = W_words.shape[1]
    nw = _NUM_CORES * _NUM_SUBCORES
    per_w = n // nw
    n_chunks = per_w // chunk
    vregs_per_chunk = chunk * d // _LANES
    d_vregs = d // _LANES

    mesh = plsc.VectorSubcoreMesh(core_axis_name="c", subcore_axis_name="s")

    @functools.partial(
        pl.kernel,
        out_type=jax.ShapeDtypeStruct((n, d), jnp.float32),
        mesh=mesh,
        scratch_types=[
            pltpu.VMEM((per_w,), jnp.int32),
            pltpu.VMEM((chunk, d), jnp.float32),
            pltpu.VMEM((chunk, d), jnp.float32),
            pltpu.SemaphoreType.DMA,
            pltpu.SemaphoreType.DMA,
        ],
    )
    def body(idx_hbm, words_hbm, pos_hbm, out_hbm, idx_v, rows_v, pos_v, gsem, psem):
        wid = lax.axis_index("s") * _NUM_CORES + lax.axis_index("c")
        base = wid * per_w
        s0 = base % seq_len
        pltpu.sync_copy(idx_hbm.at[pl.ds(base, per_w)], idx_v)
        for c in range(n_chunks):
            g = pltpu.async_copy(
                words_hbm.at[idx_v.at[pl.ds(c * chunk, chunk)]], rows_v, gsem
            )
            p = pltpu.async_copy(
                pos_hbm.at[pl.ds(s0 + c * chunk, chunk)], pos_v, psem
            )
            g.wait()
            p.wait()

            def add(i, _):
                r = i // d_vregs
                j = (i % d_vregs) * _LANES
                rows_v[r, pl.ds(j, _LANES)] = (
                    rows_v[r, pl.ds(j, _LANES)] + pos_v[r, pl.ds(j, _LANES)]
                )
                return 0

            lax.fori_loop(0, vregs_per_chunk, add, 0)
            pltpu.sync_copy(rows_v, out_hbm.at[pl.ds(base + c * chunk, chunk)])

    return body(idx, W_words, W_pos)


def kernel(input_ids, W_words, W_pos):
    b, s = input_ids.shape
    d = W_words.shape[1]
    idx = input_ids.reshape(b * s).astype(jnp.int32)
    out = _embedding_add(idx, W_words, W_pos, seq_len=s, chunk=32)
    return out.reshape(b, s, d)


# speedup vs baseline: 1.8225x; 1.8225x over previous
import functools

import jax
import jax.numpy as jnp
from jax import lax
from jax.experimental import pallas as pl
from jax.experimental.pallas import tpu as pltpu
from jax.experimental.pallas import tpu_sc as plsc

_NUM_CORES = 2
_NUM_SUBCORES = 16
_LANES = 16


@functools.partial(jax.jit, static_argnames=("seq_len", "chunk"))
def _embedding_add(idx, W_words, W_pos, seq_len, chunk):
    n = idx.shape[0]
    d = W_words.shape[1]
    nw = _NUM_CORES * _NUM_SUBCORES
    per_w = n // nw
    n_chunks = per_w // chunk
    vregs_per_chunk = chunk * d // _LANES
    d_vregs = d // _LANES

    mesh = plsc.VectorSubcoreMesh(core_axis_name="c", subcore_axis_name="s")

    @functools.partial(
        pl.kernel,
        out_type=jax.ShapeDtypeStruct((n, d), jnp.float32),
        mesh=mesh,
        scratch_types=[
            pltpu.VMEM((per_w,), jnp.int32),
            pltpu.VMEM((2, chunk, d), jnp.float32),
            pltpu.VMEM((2, chunk, d), jnp.float32),
            pltpu.SemaphoreType.DMA,
            pltpu.SemaphoreType.DMA,
            pltpu.SemaphoreType.DMA,
            pltpu.SemaphoreType.DMA,
            pltpu.SemaphoreType.DMA,
            pltpu.SemaphoreType.DMA,
        ],
    )
    def body(
        idx_hbm, words_hbm, pos_hbm, out_hbm,
        idx_v, rows_v, pos_v, g0, g1, p0, p1, o0, o1,
    ):
        gsem = (g0, g1)
        psem = (p0, p1)
        osem = (o0, o1)
        wid = lax.axis_index("s") * _NUM_CORES + lax.axis_index("c")
        base = wid * per_w
        s0 = base % seq_len
        pltpu.sync_copy(idx_hbm.at[pl.ds(base, per_w)], idx_v)

        def fetch(c, sl):
            g = pltpu.async_copy(
                words_hbm.at[idx_v.at[pl.ds(c * chunk, chunk)]],
                rows_v.at[sl],
                gsem[sl],
            )
            p = pltpu.async_copy(
                pos_hbm.at[pl.ds(s0 + c * chunk, chunk)], pos_v.at[sl], psem[sl]
            )
            return g, p

        inflight = [None, None]
        out_cp = [None, None]
        inflight[0] = fetch(0, 0)
        for c in range(n_chunks):
            sl = c % 2
            ot = 1 - sl
            if c + 1 < n_chunks:
                if out_cp[ot] is not None:
                    out_cp[ot].wait()
                    out_cp[ot] = None
                inflight[ot] = fetch(c + 1, ot)
            g, p = inflight[sl]
            g.wait()
            p.wait()

            @plsc.parallel_loop(0, vregs_per_chunk, 1, unroll=8)
            def add(i, _sl=sl):
                r = i // d_vregs
                j = (i % d_vregs) * _LANES
                rows_v[_sl, r, pl.ds(j, _LANES)] = (
                    rows_v[_sl, r, pl.ds(j, _LANES)]
                    + pos_v[_sl, r, pl.ds(j, _LANES)]
                )

            out_cp[sl] = pltpu.async_copy(
                rows_v.at[sl], out_hbm.at[pl.ds(base + c * chunk, chunk)], osem[sl]
            )
        for cp in out_cp:
            if cp is not None:
                cp.wait()

    return body(idx, W_words, W_pos)


def kernel(input_ids, W_words, W_pos):
    b, s = input_ids.shape
    d = W_words.shape[1]
    idx = input_ids.reshape(b * s).astype(jnp.int32)
    out = _embedding_add(idx, W_words, W_pos, seq_len=s, chunk=32)
    return out.reshape(b, s, d)
